# Initial kernel scaffold; baseline (speedup 1.0000x reference)
import jax
import jax.numpy as jnp
from jax.experimental import pallas as pl


def kernel(feature_maps, landmarks):
    raise NotImplementedError("write your pallas kernel here")


# speedup vs baseline: 5.8181x; 5.8181x over previous
import jax
import jax.numpy as jnp
from jax.experimental import pallas as pl
from jax.experimental.pallas import tpu as pltpu

_H = 512
_W = 512
_L = 19
_B = 4
_R1SQ = 41 * 41
_R2 = 41.0


def _tc_body(x_ref, y_ref, heat_ref, fx_ref, fy_ref, out_ref):
    b = pl.program_id(0)
    l = pl.program_id(1)

    @pl.when((b == 0) & (l == 0))
    def _init():
        out_ref[0] = 0.0
        out_ref[1] = 0.0
        out_ref[2] = 0.0
        out_ref[3] = 0.0

    X = x_ref[b, l]
    Y = y_ref[b, l]
    ii = jax.lax.broadcasted_iota(jnp.int32, (_H, _W), 0)
    jj = jax.lax.broadcasted_iota(jnp.int32, (_H, _W), 1)
    di = ii - X
    dj = jj - Y
    inside = (di * di + dj * dj) <= _R1SQ
    g = inside.astype(jnp.float32)

    p = heat_ref[0, 0]
    bce = jnp.maximum(p, 0.0) - p * g + jnp.log1p(jnp.exp(-jnp.abs(p)))
    gtx = dj.astype(jnp.float32) * (-1.0 / _R2)
    gty = di.astype(jnp.float32) * (-1.0 / _R2)
    ox = jnp.abs(fx_ref[0, 0] - gtx) * g
    oy = jnp.abs(fy_ref[0, 0] - gty) * g

    out_ref[0] += jnp.sum(bce)
    out_ref[1] += jnp.sum(g)
    out_ref[2] += jnp.sum(ox)
    out_ref[3] += jnp.sum(oy)


def kernel(feature_maps, landmarks):
    h, w = feature_maps.shape[2], feature_maps.shape[3]
    nl = feature_maps.shape[1] // 3
    X = (landmarks[:, :, 0] * (h - 1)).astype(jnp.int32)
    Y = (landmarks[:, :, 1] * (w - 1)).astype(jnp.int32)

    grid_spec = pltpu.PrefetchScalarGridSpec(
        num_scalar_prefetch=2,
        grid=(_B, _L),
        in_specs=[
            pl.BlockSpec((1, 1, _H, _W), lambda b, l, xr, yr: (b, l, 0, 0)),
            pl.BlockSpec((1, 1, _H, _W), lambda b, l, xr, yr: (b, nl + l, 0, 0)),
            pl.BlockSpec((1, 1, _H, _W), lambda b, l, xr, yr: (b, 2 * nl + l, 0, 0)),
        ],
        out_specs=pl.BlockSpec(memory_space=pltpu.SMEM),
    )
    partials = pl.pallas_call(
        _tc_body,
        grid_spec=grid_spec,
        out_shape=jax.ShapeDtypeStruct((4,), jnp.float32),
    )(X, Y, feature_maps, feature_maps, feature_maps)

    bce_sum, mask_sum, ox_sum, oy_sum = (
        partials[0], partials[1], partials[2], partials[3])
    bce = bce_sum / jnp.float32(_B * _L * _H * _W)
    denom = jnp.maximum(mask_sum, 1.0)
    return 2.0 * bce + (ox_sum + oy_sum) / denom
